# Initial kernel scaffold; baseline (speedup 1.0000x reference)
import jax
import jax.numpy as jnp
from jax.experimental import pallas as pl


def kernel(x, edge_index_1hop, W, b):
    raise NotImplementedError("write your pallas kernel here")


# speedup vs baseline: 14.9824x; 14.9824x over previous
import functools

import jax
import jax.numpy as jnp
from jax import lax
from jax.experimental import pallas as pl
from jax.experimental.pallas import tpu as pltpu
from jax.experimental.pallas import tpu_sc as plsc

NC = 2
NS = 16
CHUNK = 128
BT = 1024


def _sc_mesh():
    return plsc.VectorSubcoreMesh(core_axis_name="c", subcore_axis_name="s")


def _deg_partials(dst4, zd, n_pad, nchunk):
    rpt = n_pad // NS

    @functools.partial(
        pl.kernel,
        out_type=jax.ShapeDtypeStruct((NC, n_pad), jnp.float32),
        mesh=_sc_mesh(),
        scratch_types=[
            pltpu.VMEM((CHUNK,), jnp.int32),
            pltpu.VMEM((CHUNK,), jnp.float32),
            pltpu.VMEM_SHARED((n_pad,), jnp.float32),
        ],
    )
    def k(dst_hbm, zd_hbm, degp_hbm, dstc, ones_v, deg_sh):
        c = lax.axis_index("c")
        s = lax.axis_index("s")
        for i in range(CHUNK // 16):
            ones_v[pl.ds(i * 16, 16)] = jnp.full((16,), 1.0, jnp.float32)
        pltpu.sync_copy(zd_hbm, deg_sh.at[pl.ds(s * rpt, rpt)])
        plsc.subcore_barrier()

        def body(j, carry):
            pltpu.sync_copy(dst_hbm.at[c, s, j], dstc)
            pltpu.sync_copy(ones_v, deg_sh.at[dstc], add=True)
            return carry

        lax.fori_loop(0, nchunk, body, 0)
        plsc.subcore_barrier()
        pltpu.sync_copy(deg_sh.at[pl.ds(s * rpt, rpt)],
                        degp_hbm.at[c, pl.ds(s * rpt, rpt)])

    return k(dst4, zd)


def _scaled_linear(x_p, w, degp3, n_pad):
    d_in, d_out = w.shape

    def body(x_ref, w_ref, deg_ref, g_ref):
        h = jnp.dot(x_ref[...], w_ref[...], preferred_element_type=jnp.float32)
        deg = deg_ref[0] + deg_ref[1] + 1.0
        g_ref[...] = h * lax.rsqrt(deg)

    return pl.pallas_call(
        body,
        grid=(n_pad // BT,),
        in_specs=[
            pl.BlockSpec((BT, d_in), lambda i: (i, 0)),
            pl.BlockSpec((d_in, d_out), lambda i: (0, 0)),
            pl.BlockSpec((NC, BT, 1), lambda i: (0, i, 0)),
        ],
        out_specs=pl.BlockSpec((BT, d_out), lambda i: (i, 0)),
        out_shape=jax.ShapeDtypeStruct((n_pad, d_out), jnp.float32),
    )(x_p, w, degp3)


def _gather_scatter(src4, dst4, g, z, n_pad, nchunk):
    d = g.shape[1]
    rpt = n_pad // NS

    @functools.partial(
        pl.kernel,
        out_type=jax.ShapeDtypeStruct((NC, n_pad, d), jnp.float32),
        mesh=_sc_mesh(),
        scratch_types=[
            pltpu.VMEM((CHUNK,), jnp.int32),
            pltpu.VMEM((CHUNK,), jnp.int32),
            pltpu.VMEM((CHUNK, d), jnp.float32),
            pltpu.VMEM_SHARED((n_pad, d), jnp.float32),
            pltpu.SemaphoreType.DMA,
        ],
    )
    def k(src_hbm, dst_hbm, g_hbm, z_hbm, acc_hbm, srcc, dstc, rows, acc_sh, sem):
        c = lax.axis_index("c")
        s = lax.axis_index("s")
        pltpu.sync_copy(z_hbm, acc_sh.at[pl.ds(s * rpt, rpt)])
        plsc.subcore_barrier()

        def body(j, carry):
            pltpu.sync_copy(src_hbm.at[c, s, j], srcc)
            pltpu.sync_copy(dst_hbm.at[c, s, j], dstc)
            pltpu.async_copy(g_hbm.at[srcc], rows, sem).wait()
            pltpu.sync_copy(rows, acc_sh.at[dstc], add=True)
            return carry

        lax.fori_loop(0, nchunk, body, 0)
        plsc.subcore_barrier()
        pltpu.sync_copy(acc_sh.at[pl.ds(s * rpt, rpt)],
                        acc_hbm.at[c, pl.ds(s * rpt, rpt)])

    return k(src4, dst4, g, z)


def _finalize(acc, g, degp3, b2, n_pad):
    d_out = g.shape[1]

    def body(acc_ref, g_ref, deg_ref, b_ref, out_ref):
        deg = deg_ref[0] + deg_ref[1] + 1.0
        dis = lax.rsqrt(deg)
        tot = acc_ref[0] + acc_ref[1] + g_ref[...]
        out_ref[...] = jnp.maximum(tot * dis + b_ref[...], 0.0)

    return pl.pallas_call(
        body,
        grid=(n_pad // BT,),
        in_specs=[
            pl.BlockSpec((NC, BT, d_out), lambda i: (0, i, 0)),
            pl.BlockSpec((BT, d_out), lambda i: (i, 0)),
            pl.BlockSpec((NC, BT, 1), lambda i: (0, i, 0)),
            pl.BlockSpec((1, d_out), lambda i: (0, 0)),
        ],
        out_specs=pl.BlockSpec((BT, d_out), lambda i: (i, 0)),
        out_shape=jax.ShapeDtypeStruct((n_pad, d_out), jnp.float32),
    )(acc, g, degp3, b2)


def kernel(x, edge_index_1hop, W, b):
    n, d_in = x.shape
    d_out = W.shape[1]
    e = edge_index_1hop.shape[1]

    n_pad = ((n + BT - 1) // BT) * BT
    ec = NC * NS * CHUNK
    e_pad = ((e + ec - 1) // ec) * ec
    nchunk = e_pad // ec
    rpt = n_pad // NS

    src = edge_index_1hop[0]
    dst = edge_index_1hop[1]
    pad_e = e_pad - e
    src_p = jnp.concatenate([src, jnp.zeros((pad_e,), jnp.int32)])
    dst_p = jnp.concatenate([dst, jnp.full((pad_e,), n, jnp.int32)])
    src4 = src_p.reshape(NC, NS, nchunk, CHUNK)
    dst4 = dst_p.reshape(NC, NS, nchunk, CHUNK)

    x_p = jnp.pad(x, ((0, n_pad - n), (0, 0)))
    zd = jnp.zeros((rpt,), jnp.float32)
    z = jnp.zeros((rpt, d_out), jnp.float32)

    degp = _deg_partials(dst4, zd, n_pad, nchunk)
    degp3 = degp[:, :, None]
    g = _scaled_linear(x_p, W, degp3, n_pad)
    acc = _gather_scatter(src4, dst4, g, z, n_pad, nchunk)
    out = _finalize(acc, g, degp3, b.reshape(1, d_out), n_pad)
    return out[:n]
